# baseline (speedup 1.0000x reference)
import functools

import jax
import jax.numpy as jnp
from jax import lax
from jax.experimental import pallas as pl
from jax.experimental.pallas import tpu as pltpu
from jax.experimental.pallas import tpu_sc as plsc

N = 10000
D = 128
E = 320000
NC, NS = 2, 16
NW = NC * NS
K = 80
NCHUNK = 128
EPW = NCHUNK * K
EP = NW * EPW
NP = 10112
SINK = NP - 1
ROWS_PT = NP // NS
DEG_N = 10240
DEG_PT = DEG_N // NS

_MESH = dict(core_axis_name="c", subcore_axis_name="s")


@functools.partial(
    pl.kernel,
    out_type=jax.ShapeDtypeStruct((NC, DEG_N), jnp.float32),
    mesh=plsc.VectorSubcoreMesh(**_MESH),
    scratch_types=[
        pltpu.VMEM((8, K), jnp.int32),
        pltpu.VMEM((2, K), jnp.float32),
        pltpu.VMEM_SHARED((DEG_N,), jnp.float32),
    ],
)
def _deg_kernel(dst_hbm, out_hbm, dst_v, ones_v, deg_sh):
    c = lax.axis_index("c")
    s = lax.axis_index("s")
    w = c * NS + s

    for i in range(K // 16):
        ones_v[0, pl.ds(i * 16, 16)] = jnp.ones((16,), jnp.float32)
        ones_v[1, pl.ds(i * 16, 16)] = jnp.zeros((16,), jnp.float32)

    for r in range(DEG_PT // K):
        pltpu.sync_copy(ones_v.at[1],
                        deg_sh.at[pl.ds(s * DEG_PT + r * K, K)])
    plsc.subcore_barrier()

    def _slab(j, _):
        pltpu.sync_copy(dst_hbm.at[w, pl.ds(j * 8, 8)], dst_v)
        for r in range(8):
            pltpu.sync_copy(ones_v.at[0], deg_sh.at[dst_v.at[r]], add=True)
        return 0

    lax.fori_loop(0, NCHUNK // 8, _slab, 0)
    plsc.subcore_barrier()
    pltpu.sync_copy(
        deg_sh.at[pl.ds(s * DEG_PT, DEG_PT)],
        out_hbm.at[c, pl.ds(s * DEG_PT, DEG_PT)],
    )


@functools.partial(
    pl.kernel,
    out_type=jax.ShapeDtypeStruct((NC, NP, D), jnp.float32),
    mesh=plsc.VectorSubcoreMesh(**_MESH),
    scratch_types=[
        pltpu.VMEM((EPW,), jnp.int32),
        pltpu.VMEM((NCHUNK, K), jnp.int32),
        pltpu.VMEM((K, D), jnp.float32),
        pltpu.VMEM((K, D), jnp.float32),
        pltpu.VMEM_SHARED((NP, D), jnp.float32),
        pltpu.SemaphoreType.DMA,
        pltpu.SemaphoreType.DMA,
        pltpu.SemaphoreType.DMA,
        pltpu.SemaphoreType.DMA,
    ],
)
def _agg_kernel(g_hbm, src_hbm, dst_hbm, out_hbm,
                src_all, dst_all, rows0, rows1, agg_sh,
                sg0, sg1, ss0, ss1):
    c = lax.axis_index("c")
    s = lax.axis_index("s")
    w = c * NS + s

    pltpu.sync_copy(src_hbm.at[w], src_all)
    pltpu.sync_copy(dst_hbm.at[w], dst_all)

    def _z(i, _):
        rows0[i // 8, pl.ds((i % 8) * 16, 16)] = jnp.zeros((16,), jnp.float32)
        return 0

    lax.fori_loop(0, K * 8, _z, 0)
    base = s * ROWS_PT
    for r in range(ROWS_PT // K):
        pltpu.sync_copy(rows0, agg_sh.at[pl.ds(base + r * K, K)])
    rem = ROWS_PT % K
    pltpu.sync_copy(rows0.at[pl.ds(0, rem)],
                    agg_sh.at[pl.ds(base + ROWS_PT - rem, rem)])
    plsc.subcore_barrier()

    def gather_start(j, rows, sem):
        pltpu.async_copy(g_hbm.at[src_all.at[pl.ds(j * K, K)]], rows, sem)

    def gather_wait(j, rows, sem):
        pltpu.make_async_copy(
            g_hbm.at[src_all.at[pl.ds(j * K, K)]], rows, sem).wait()

    def scatter_start(j, rows, sem):
        pltpu.async_copy(rows, agg_sh.at[dst_all.at[j]], sem, add=True)

    def scatter_wait(j, rows, sem):
        pltpu.make_async_copy(rows, agg_sh.at[dst_all.at[j]], sem).wait()

    gather_start(0, rows0, sg0)
    gather_start(1, rows1, sg1)

    def _step(t, _):
        a = 2 * t
        b = a + 1
        gather_wait(a, rows0, sg0)
        scatter_start(a, rows0, ss0)
        gather_wait(b, rows1, sg1)
        scatter_start(b, rows1, ss1)
        scatter_wait(a, rows0, ss0)
        gather_start(a + 2, rows0, sg0)

        @pl.when(t < NCHUNK // 2 - 2)
        def _():
            scatter_wait(b, rows1, ss1)
            gather_start(b + 2, rows1, sg1)

        return 0

    lax.fori_loop(0, NCHUNK // 2 - 1, _step, 0)

    gather_wait(NCHUNK - 2, rows0, sg0)
    scatter_start(NCHUNK - 2, rows0, ss0)
    scatter_wait(NCHUNK - 3, rows1, ss1)
    gather_start(NCHUNK - 1, rows1, sg1)
    gather_wait(NCHUNK - 1, rows1, sg1)
    scatter_start(NCHUNK - 1, rows1, ss1)
    scatter_wait(NCHUNK - 2, rows0, ss0)
    scatter_wait(NCHUNK - 1, rows1, ss1)

    plsc.subcore_barrier()
    pltpu.sync_copy(
        agg_sh.at[pl.ds(s * ROWS_PT, ROWS_PT)],
        out_hbm.at[c, pl.ds(s * ROWS_PT, ROWS_PT)],
    )


_R = 632
_RF = 1000


def _dinv(degA_ref, degB_ref):
    return lax.rsqrt(degA_ref[...] + degB_ref[...] + 1.0)


def _mm_scale_body(x_ref, w_ref, degA_ref, degB_ref, o_ref):
    h = jnp.dot(x_ref[...], w_ref[...], preferred_element_type=jnp.float32)
    o_ref[...] = h * _dinv(degA_ref, degB_ref)


def _layer2_body(g1_ref, aggA_ref, aggB_ref, degA_ref, degB_ref,
                 w2_ref, b1_ref, o_ref):
    dinv = _dinv(degA_ref, degB_ref)
    h = dinv * (aggA_ref[...] + aggB_ref[...] + g1_ref[...]) + b1_ref[...]
    h = jnp.maximum(h, 0.0)
    o_ref[...] = jnp.dot(h, w2_ref[...],
                         preferred_element_type=jnp.float32) * dinv


def _final_body(g2_ref, aggA_ref, aggB_ref, degA_ref, degB_ref,
                b2_ref, o_ref):
    dinv = _dinv(degA_ref, degB_ref)
    o_ref[...] = dinv * (aggA_ref[...] + aggB_ref[...] + g2_ref[...]) + b2_ref[...]


def _row_spec(r, width=D):
    return pl.BlockSpec((r, width), lambda i: (i, 0))


def _full_spec(shape):
    return pl.BlockSpec(shape, lambda i: (0, 0))


def _mm_scale(xp, W, degA, degB):
    return pl.pallas_call(
        _mm_scale_body,
        grid=(NP // _R,),
        in_specs=[_row_spec(_R), _full_spec((D, D)), _row_spec(_R, 1),
                  _row_spec(_R, 1)],
        out_specs=_row_spec(_R),
        out_shape=jax.ShapeDtypeStruct((NP, D), jnp.float32),
    )(xp, W, degA, degB)


def _layer2(g1, aggA, aggB, degA, degB, W2, b1):
    return pl.pallas_call(
        _layer2_body,
        grid=(NP // _R,),
        in_specs=[_row_spec(_R), _row_spec(_R), _row_spec(_R),
                  _row_spec(_R, 1), _row_spec(_R, 1),
                  _full_spec((D, D)), _full_spec((1, D))],
        out_specs=_row_spec(_R),
        out_shape=jax.ShapeDtypeStruct((NP, D), jnp.float32),
    )(g1, aggA, aggB, degA, degB, W2, b1)


def _final(g2, aggA, aggB, degA, degB, b2):
    return pl.pallas_call(
        _final_body,
        grid=(N // _RF,),
        in_specs=[_row_spec(_RF), _row_spec(_RF), _row_spec(_RF),
                  _row_spec(_RF, 1), _row_spec(_RF, 1), _full_spec((1, D))],
        out_specs=_row_spec(_RF),
        out_shape=jax.ShapeDtypeStruct((N, D), jnp.float32),
    )(g2, aggA, aggB, degA, degB, b2)


def kernel(x, edge_index, W1, b1, W2, b2):
    src = edge_index[0].astype(jnp.int32)
    dst = edge_index[1].astype(jnp.int32)
    pad = jnp.full((EP - E,), SINK, dtype=jnp.int32)
    src3 = jnp.concatenate([src, pad]).reshape(NW, EPW)
    dst3 = jnp.concatenate([dst, pad]).reshape(NW, NCHUNK, K)
    xp = jnp.pad(x, ((0, NP - N), (0, 0)))
    b1r = b1.reshape(1, D)
    b2r = b2.reshape(1, D)

    deg_parts = _deg_kernel(dst3)
    degA = deg_parts[0, :NP].reshape(NP, 1)
    degB = deg_parts[1, :NP].reshape(NP, 1)

    g1 = _mm_scale(xp, W1, degA, degB)
    agg1 = _agg_kernel(g1, src3, dst3)
    g2 = _layer2(g1, agg1[0], agg1[1], degA, degB, W2, b1r)
    agg2 = _agg_kernel(g2, src3, dst3)
    return _final(g2, agg2[0, :N], agg2[1, :N], degA[:N], degB[:N], b2r)


# speedup vs baseline: 1.0024x; 1.0024x over previous
import functools

import jax
import jax.numpy as jnp
from jax import lax
from jax.experimental import pallas as pl
from jax.experimental.pallas import tpu as pltpu
from jax.experimental.pallas import tpu_sc as plsc

N = 10000
D = 128
E = 320000
NC, NS = 2, 16
NW = NC * NS
K = 80
NCHUNK = 128
EPW = NCHUNK * K
EP = NW * EPW
NP = 10112
SINK = NP - 1
ROWS_PT = NP // NS
DEG_N = 10240
DEG_PT = DEG_N // NS

_MESH = dict(core_axis_name="c", subcore_axis_name="s")


@functools.partial(
    pl.kernel,
    out_type=jax.ShapeDtypeStruct((NC, DEG_N), jnp.float32),
    mesh=plsc.VectorSubcoreMesh(**_MESH),
    scratch_types=[
        pltpu.VMEM((8, K), jnp.int32),
        pltpu.VMEM((2, K), jnp.float32),
        pltpu.VMEM_SHARED((DEG_N,), jnp.float32),
    ],
)
def _deg_kernel(dst_hbm, out_hbm, dst_v, ones_v, deg_sh):
    c = lax.axis_index("c")
    s = lax.axis_index("s")
    w = c * NS + s

    for i in range(K // 16):
        ones_v[0, pl.ds(i * 16, 16)] = jnp.ones((16,), jnp.float32)
        ones_v[1, pl.ds(i * 16, 16)] = jnp.zeros((16,), jnp.float32)

    for r in range(DEG_PT // K):
        pltpu.sync_copy(ones_v.at[1],
                        deg_sh.at[pl.ds(s * DEG_PT + r * K, K)])
    plsc.subcore_barrier()

    def _slab(j, _):
        pltpu.sync_copy(dst_hbm.at[w, pl.ds(j * 8, 8)], dst_v)
        for r in range(8):
            pltpu.sync_copy(ones_v.at[0], deg_sh.at[dst_v.at[r]], add=True)
        return 0

    lax.fori_loop(0, NCHUNK // 8, _slab, 0)
    plsc.subcore_barrier()
    pltpu.sync_copy(
        deg_sh.at[pl.ds(s * DEG_PT, DEG_PT)],
        out_hbm.at[c, pl.ds(s * DEG_PT, DEG_PT)],
    )


@functools.partial(
    pl.kernel,
    out_type=jax.ShapeDtypeStruct((NC, NP, D), jnp.float32),
    mesh=plsc.VectorSubcoreMesh(**_MESH),
    scratch_types=[
        pltpu.VMEM((EPW,), jnp.int32),
        pltpu.VMEM((NCHUNK, K), jnp.int32),
        pltpu.VMEM((K, D), jnp.float32),
        pltpu.VMEM((K, D), jnp.float32),
        pltpu.VMEM_SHARED((NP, D), jnp.float32),
        pltpu.SemaphoreType.DMA,
        pltpu.SemaphoreType.DMA,
        pltpu.SemaphoreType.DMA,
        pltpu.SemaphoreType.DMA,
    ],
)
def _agg_kernel(g_hbm, src_hbm, dst_hbm, out_hbm,
                src_all, dst_all, rows0, rows1, agg_sh,
                sg0, sg1, ss0, ss1):
    c = lax.axis_index("c")
    s = lax.axis_index("s")
    w = c * NS + s

    pltpu.sync_copy(src_hbm.at[w], src_all)
    pltpu.sync_copy(dst_hbm.at[w], dst_all)

    def _z(i, _):
        rows0[i // 8, pl.ds((i % 8) * 16, 16)] = jnp.zeros((16,), jnp.float32)
        return 0

    lax.fori_loop(0, K * 8, _z, 0)
    base = s * ROWS_PT
    for r in range(ROWS_PT // K):
        pltpu.sync_copy(rows0, agg_sh.at[pl.ds(base + r * K, K)])
    rem = ROWS_PT % K
    pltpu.sync_copy(rows0.at[pl.ds(0, rem)],
                    agg_sh.at[pl.ds(base + ROWS_PT - rem, rem)])
    plsc.subcore_barrier()

    def gather_start(j, rows, sem):
        pltpu.async_copy(g_hbm.at[src_all.at[pl.ds(j * K, K)]], rows, sem)

    def gather_wait(j, rows, sem):
        pltpu.make_async_copy(
            g_hbm.at[src_all.at[pl.ds(j * K, K)]], rows, sem).wait()

    def scatter_start(j, rows, sem):
        pltpu.async_copy(rows, agg_sh.at[dst_all.at[j]], sem, add=True)

    def scatter_wait(j, rows, sem):
        pltpu.make_async_copy(rows, agg_sh.at[dst_all.at[j]], sem).wait()

    gather_start(0, rows0, sg0)
    gather_start(1, rows1, sg1)

    def _step(t, _):
        a = 2 * t
        b = a + 1
        gather_wait(a, rows0, sg0)
        scatter_start(a, rows0, ss0)
        gather_wait(b, rows1, sg1)
        scatter_start(b, rows1, ss1)
        scatter_wait(a, rows0, ss0)
        gather_start(a + 2, rows0, sg0)

        @pl.when(t < NCHUNK // 2 - 2)
        def _():
            scatter_wait(b, rows1, ss1)
            gather_start(b + 2, rows1, sg1)

        return 0

    lax.fori_loop(0, NCHUNK // 2 - 1, _step, 0)

    gather_wait(NCHUNK - 2, rows0, sg0)
    scatter_start(NCHUNK - 2, rows0, ss0)
    scatter_wait(NCHUNK - 3, rows1, ss1)
    gather_start(NCHUNK - 1, rows1, sg1)
    gather_wait(NCHUNK - 1, rows1, sg1)
    scatter_start(NCHUNK - 1, rows1, ss1)
    scatter_wait(NCHUNK - 2, rows0, ss0)
    scatter_wait(NCHUNK - 1, rows1, ss1)

    plsc.subcore_barrier()
    pltpu.sync_copy(
        agg_sh.at[pl.ds(s * ROWS_PT, ROWS_PT)],
        out_hbm.at[c, pl.ds(s * ROWS_PT, ROWS_PT)],
    )


_R = 632
_RF = 1000


def _dinv(degA_ref, degB_ref):
    return lax.rsqrt(degA_ref[...] + degB_ref[...] + 1.0)


def _mm_scale_body(x_ref, w_ref, degA_ref, degB_ref, o_ref):
    h = jnp.dot(x_ref[...], w_ref[...], preferred_element_type=jnp.float32)
    o_ref[...] = h * _dinv(degA_ref, degB_ref)


def _layer2_body(g1_ref, aggA_ref, aggB_ref, degA_ref, degB_ref,
                 w2_ref, b1_ref, o_ref):
    dinv = _dinv(degA_ref, degB_ref)
    h = dinv * (aggA_ref[...] + aggB_ref[...] + g1_ref[...]) + b1_ref[...]
    h = jnp.maximum(h, 0.0)
    o_ref[...] = jnp.dot(h, w2_ref[...],
                         preferred_element_type=jnp.float32) * dinv


def _final_body(g2_ref, aggA_ref, aggB_ref, degA_ref, degB_ref,
                b2_ref, o_ref):
    dinv = _dinv(degA_ref, degB_ref)
    o_ref[...] = dinv * (aggA_ref[...] + aggB_ref[...] + g2_ref[...]) + b2_ref[...]


def _row_spec(r, width=D):
    return pl.BlockSpec((r, width), lambda i: (i, 0))


def _full_spec(shape):
    return pl.BlockSpec(shape, lambda i: (0, 0))


def _mm_scale(xp, W, degA, degB):
    return pl.pallas_call(
        _mm_scale_body,
        grid=(NP // _R,),
        in_specs=[_row_spec(_R), _full_spec((D, D)), _row_spec(_R, 1),
                  _row_spec(_R, 1)],
        out_specs=_row_spec(_R),
        out_shape=jax.ShapeDtypeStruct((NP, D), jnp.float32),
    )(xp, W, degA, degB)


def _layer2(g1, aggA, aggB, degA, degB, W2, b1):
    return pl.pallas_call(
        _layer2_body,
        grid=(NP // _R,),
        in_specs=[_row_spec(_R), _row_spec(_R), _row_spec(_R),
                  _row_spec(_R, 1), _row_spec(_R, 1),
                  _full_spec((D, D)), _full_spec((1, D))],
        out_specs=_row_spec(_R),
        out_shape=jax.ShapeDtypeStruct((NP, D), jnp.float32),
    )(g1, aggA, aggB, degA, degB, W2, b1)


def _final(g2, aggA, aggB, degA, degB, b2):
    return pl.pallas_call(
        _final_body,
        grid=(N // _RF,),
        in_specs=[_row_spec(_RF), _row_spec(_RF), _row_spec(_RF),
                  _row_spec(_RF, 1), _row_spec(_RF, 1), _full_spec((1, D))],
        out_specs=_row_spec(_RF),
        out_shape=jax.ShapeDtypeStruct((N, D), jnp.float32),
    )(g2, aggA, aggB, degA, degB, b2)


def kernel(x, edge_index, W1, b1, W2, b2):
    src = edge_index[0].astype(jnp.int32)
    dst = edge_index[1].astype(jnp.int32)
    pad_src = jnp.full((EP - E,), SINK, dtype=jnp.int32)
    pad_dst = N + (jnp.arange(EP - E, dtype=jnp.int32) % (NP - N))
    src3 = jnp.concatenate([src, pad_src]).reshape(NW, EPW)
    dst3 = jnp.concatenate([dst, pad_dst]).reshape(NW, NCHUNK, K)
    xp = jnp.pad(x, ((0, NP - N), (0, 0)))
    b1r = b1.reshape(1, D)
    b2r = b2.reshape(1, D)

    deg_parts = _deg_kernel(dst3)
    degA = deg_parts[0, :NP].reshape(NP, 1)
    degB = deg_parts[1, :NP].reshape(NP, 1)

    g1 = _mm_scale(xp, W1, degA, degB)
    agg1 = _agg_kernel(g1, src3, dst3)
    g2 = _layer2(g1, agg1[0], agg1[1], degA, degB, W2, b1r)
    agg2 = _agg_kernel(g2, src3, dst3)
    return _final(g2, agg2[0, :N], agg2[1, :N], degA[:N], degB[:N], b2r)


# speedup vs baseline: 2.3698x; 2.3642x over previous
import functools

import jax
import jax.numpy as jnp
from jax import lax
from jax.experimental import pallas as pl
from jax.experimental.pallas import tpu as pltpu
from jax.experimental.pallas import tpu_sc as plsc

N = 10000
D = 128
E = 320000
NC, NS = 2, 16
NW = NC * NS
K = 80
NCHUNK = 128
EPW = NCHUNK * K
EP = NW * EPW
NP = 10112
SINK = NP - 1
ROWS_PT = NP // NS
DEG_N = 10240
DEG_PT = DEG_N // NS

_MESH = dict(core_axis_name="c", subcore_axis_name="s")


@functools.partial(
    pl.kernel,
    out_type=jax.ShapeDtypeStruct((NC, DEG_N), jnp.float32),
    mesh=plsc.VectorSubcoreMesh(**_MESH),
    scratch_types=[
        pltpu.VMEM((8, K), jnp.int32),
        pltpu.VMEM((2, K), jnp.float32),
        pltpu.VMEM_SHARED((DEG_N,), jnp.float32),
    ],
)
def _deg_kernel(dst_hbm, out_hbm, dst_v, ones_v, deg_sh):
    c = lax.axis_index("c")
    s = lax.axis_index("s")
    w = c * NS + s

    for i in range(K // 16):
        ones_v[0, pl.ds(i * 16, 16)] = jnp.ones((16,), jnp.float32)
        ones_v[1, pl.ds(i * 16, 16)] = jnp.zeros((16,), jnp.float32)

    for r in range(DEG_PT // K):
        pltpu.sync_copy(ones_v.at[1],
                        deg_sh.at[pl.ds(s * DEG_PT + r * K, K)])
    plsc.subcore_barrier()

    def _slab(j, _):
        pltpu.sync_copy(dst_hbm.at[w, pl.ds(j * 8, 8)], dst_v)
        for r in range(8):
            pltpu.sync_copy(ones_v.at[0], deg_sh.at[dst_v.at[r]], add=True)
        return 0

    lax.fori_loop(0, NCHUNK // 8, _slab, 0)
    plsc.subcore_barrier()
    pltpu.sync_copy(
        deg_sh.at[pl.ds(s * DEG_PT, DEG_PT)],
        out_hbm.at[c, pl.ds(s * DEG_PT, DEG_PT)],
    )


@functools.partial(
    pl.kernel,
    out_type=jax.ShapeDtypeStruct((NC, NP, D), jnp.float32),
    mesh=plsc.VectorSubcoreMesh(**_MESH),
    scratch_types=[
        pltpu.VMEM((EPW,), jnp.int32),
        pltpu.VMEM((NCHUNK, K), jnp.int32),
        pltpu.VMEM((K, D), jnp.float32),
        pltpu.VMEM((K, D), jnp.float32),
        pltpu.VMEM_SHARED((NP, D), jnp.float32),
        pltpu.SemaphoreType.DMA,
        pltpu.SemaphoreType.DMA,
        pltpu.SemaphoreType.DMA,
        pltpu.SemaphoreType.DMA,
    ],
)
def _agg_kernel(g_hbm, src_hbm, dst_hbm, out_hbm,
                src_all, dst_all, rows0, rows1, agg_sh,
                sg0, sg1, ss0, ss1):
    c = lax.axis_index("c")
    s = lax.axis_index("s")
    w = c * NS + s

    pltpu.sync_copy(src_hbm.at[w], src_all)
    pltpu.sync_copy(dst_hbm.at[w], dst_all)

    def _z(i, _):
        rows0[i // 8, pl.ds((i % 8) * 16, 16)] = jnp.zeros((16,), jnp.float32)
        return 0

    lax.fori_loop(0, K * 8, _z, 0)
    base = s * ROWS_PT
    for r in range(ROWS_PT // K):
        pltpu.sync_copy(rows0, agg_sh.at[pl.ds(base + r * K, K)])
    rem = ROWS_PT % K
    pltpu.sync_copy(rows0.at[pl.ds(0, rem)],
                    agg_sh.at[pl.ds(base + ROWS_PT - rem, rem)])
    plsc.subcore_barrier()

    def gather_start(j, rows, sem):
        pltpu.async_copy(g_hbm.at[src_all.at[pl.ds(j * K, K)]], rows, sem)

    def gather_wait(j, rows, sem):
        pltpu.make_async_copy(
            g_hbm.at[src_all.at[pl.ds(j * K, K)]], rows, sem).wait()

    def scatter_start(j, rows, sem):
        pltpu.async_copy(rows, agg_sh.at[dst_all.at[j]], sem, add=True)

    def scatter_wait(j, rows, sem):
        pltpu.make_async_copy(rows, agg_sh.at[dst_all.at[j]], sem).wait()

    gather_start(0, rows0, sg0)
    gather_start(1, rows1, sg1)

    def _step(t, _):
        a = 2 * t
        b = a + 1
        gather_wait(a, rows0, sg0)
        scatter_start(a, rows0, ss0)
        gather_wait(b, rows1, sg1)
        scatter_start(b, rows1, ss1)
        scatter_wait(a, rows0, ss0)
        gather_start(a + 2, rows0, sg0)

        @pl.when(t < NCHUNK // 2 - 2)
        def _():
            scatter_wait(b, rows1, ss1)
            gather_start(b + 2, rows1, sg1)

        return 0

    lax.fori_loop(0, NCHUNK // 2 - 1, _step, 0)

    gather_wait(NCHUNK - 2, rows0, sg0)
    scatter_start(NCHUNK - 2, rows0, ss0)
    scatter_wait(NCHUNK - 3, rows1, ss1)
    gather_start(NCHUNK - 1, rows1, sg1)
    gather_wait(NCHUNK - 1, rows1, sg1)
    scatter_start(NCHUNK - 1, rows1, ss1)
    scatter_wait(NCHUNK - 2, rows0, ss0)
    scatter_wait(NCHUNK - 1, rows1, ss1)

    plsc.subcore_barrier()
    pltpu.sync_copy(
        agg_sh.at[pl.ds(s * ROWS_PT, ROWS_PT)],
        out_hbm.at[c, pl.ds(s * ROWS_PT, ROWS_PT)],
    )


_R = 632
_RF = 1000


def _dinv(degA_ref, degB_ref):
    return lax.rsqrt(degA_ref[...] + degB_ref[...] + 1.0)


def _mm_scale_body(x_ref, w_ref, degA_ref, degB_ref, o_ref):
    h = jnp.dot(x_ref[...], w_ref[...], preferred_element_type=jnp.float32)
    o_ref[...] = h * _dinv(degA_ref, degB_ref)


def _layer2_body(g1_ref, aggA_ref, aggB_ref, degA_ref, degB_ref,
                 w2_ref, b1_ref, o_ref):
    dinv = _dinv(degA_ref, degB_ref)
    h = dinv * (aggA_ref[...] + aggB_ref[...] + g1_ref[...]) + b1_ref[...]
    h = jnp.maximum(h, 0.0)
    o_ref[...] = jnp.dot(h, w2_ref[...],
                         preferred_element_type=jnp.float32) * dinv


def _final_body(g2_ref, aggA_ref, aggB_ref, degA_ref, degB_ref,
                b2_ref, o_ref):
    dinv = _dinv(degA_ref, degB_ref)
    o_ref[...] = dinv * (aggA_ref[...] + aggB_ref[...] + g2_ref[...]) + b2_ref[...]


def _row_spec(r, width=D):
    return pl.BlockSpec((r, width), lambda i: (i, 0))


def _full_spec(shape):
    return pl.BlockSpec(shape, lambda i: (0, 0))


def _mm_scale(xp, W, degA, degB):
    return pl.pallas_call(
        _mm_scale_body,
        grid=(NP // _R,),
        in_specs=[_row_spec(_R), _full_spec((D, D)), _row_spec(_R, 1),
                  _row_spec(_R, 1)],
        out_specs=_row_spec(_R),
        out_shape=jax.ShapeDtypeStruct((NP, D), jnp.float32),
    )(xp, W, degA, degB)


def _layer2(g1, aggA, aggB, degA, degB, W2, b1):
    return pl.pallas_call(
        _layer2_body,
        grid=(NP // _R,),
        in_specs=[_row_spec(_R), _row_spec(_R), _row_spec(_R),
                  _row_spec(_R, 1), _row_spec(_R, 1),
                  _full_spec((D, D)), _full_spec((1, D))],
        out_specs=_row_spec(_R),
        out_shape=jax.ShapeDtypeStruct((NP, D), jnp.float32),
    )(g1, aggA, aggB, degA, degB, W2, b1)


def _final(g2, aggA, aggB, degA, degB, b2):
    return pl.pallas_call(
        _final_body,
        grid=(N // _RF,),
        in_specs=[_row_spec(_RF), _row_spec(_RF), _row_spec(_RF),
                  _row_spec(_RF, 1), _row_spec(_RF, 1), _full_spec((1, D))],
        out_specs=_row_spec(_RF),
        out_shape=jax.ShapeDtypeStruct((N, D), jnp.float32),
    )(g2, aggA, aggB, degA, degB, b2)


def kernel(x, edge_index, W1, b1, W2, b2):
    src = edge_index[0].astype(jnp.int32)
    dst = edge_index[1].astype(jnp.int32)
    pad_idx = N + (jnp.arange(EP - E, dtype=jnp.int32) % (NP - N))
    src3 = jnp.concatenate([src, pad_idx]).reshape(NW, EPW)
    dst3 = jnp.concatenate([dst, pad_idx]).reshape(NW, NCHUNK, K)
    xp = jnp.pad(x, ((0, NP - N), (0, 0)))
    b1r = b1.reshape(1, D)
    b2r = b2.reshape(1, D)

    deg_parts = _deg_kernel(dst3)
    degA = deg_parts[0, :NP].reshape(NP, 1)
    degB = deg_parts[1, :NP].reshape(NP, 1)

    g1 = _mm_scale(xp, W1, degA, degB)
    agg1 = _agg_kernel(g1, src3, dst3)
    g2 = _layer2(g1, agg1[0], agg1[1], degA, degB, W2, b1r)
    agg2 = _agg_kernel(g2, src3, dst3)
    return _final(g2, agg2[0, :N], agg2[1, :N], degA[:N], degB[:N], b2r)


# speedup vs baseline: 2.3736x; 1.0016x over previous
import functools

import jax
import jax.numpy as jnp
from jax import lax
from jax.experimental import pallas as pl
from jax.experimental.pallas import tpu as pltpu
from jax.experimental.pallas import tpu_sc as plsc

N = 10000
D = 128
E = 320000
NC, NS = 2, 16
NW = NC * NS
K = 80
NCHUNK = 128
EPW = NCHUNK * K
EP = NW * EPW
NP = 10112
SINK = NP - 1
ROWS_PT = NP // NS
DEG_N = 10240
DEG_PT = DEG_N // NS

_MESH = dict(core_axis_name="c", subcore_axis_name="s")


@functools.partial(
    pl.kernel,
    out_type=jax.ShapeDtypeStruct((NC, DEG_N), jnp.float32),
    mesh=plsc.VectorSubcoreMesh(**_MESH),
    scratch_types=[
        pltpu.VMEM((8, K), jnp.int32),
        pltpu.VMEM((2, K), jnp.float32),
        pltpu.VMEM_SHARED((DEG_N,), jnp.float32),
    ],
)
def _deg_kernel(dst_hbm, out_hbm, dst_v, ones_v, deg_sh):
    c = lax.axis_index("c")
    s = lax.axis_index("s")
    w = c * NS + s

    for i in range(K // 16):
        ones_v[0, pl.ds(i * 16, 16)] = jnp.ones((16,), jnp.float32)
        ones_v[1, pl.ds(i * 16, 16)] = jnp.zeros((16,), jnp.float32)

    for r in range(DEG_PT // K):
        pltpu.sync_copy(ones_v.at[1],
                        deg_sh.at[pl.ds(s * DEG_PT + r * K, K)])
    plsc.subcore_barrier()

    def _slab(j, _):
        pltpu.sync_copy(dst_hbm.at[w, pl.ds(j * 8, 8)], dst_v)
        for r in range(8):
            pltpu.sync_copy(ones_v.at[0], deg_sh.at[dst_v.at[r]], add=True)
        return 0

    lax.fori_loop(0, NCHUNK // 8, _slab, 0)
    plsc.subcore_barrier()
    pltpu.sync_copy(
        deg_sh.at[pl.ds(s * DEG_PT, DEG_PT)],
        out_hbm.at[c, pl.ds(s * DEG_PT, DEG_PT)],
    )


@functools.partial(
    pl.kernel,
    out_type=jax.ShapeDtypeStruct((NC, NP, D), jnp.float32),
    mesh=plsc.VectorSubcoreMesh(**_MESH),
    scratch_types=[
        pltpu.VMEM((EPW,), jnp.int32),
        pltpu.VMEM((NCHUNK, K), jnp.int32),
        pltpu.VMEM((K, D), jnp.float32),
        pltpu.VMEM((K, D), jnp.float32),
        pltpu.VMEM_SHARED((NP, D), jnp.float32),
        pltpu.SemaphoreType.DMA,
        pltpu.SemaphoreType.DMA,
        pltpu.SemaphoreType.DMA,
        pltpu.SemaphoreType.DMA,
    ],
)
def _agg_kernel(g_hbm, src_hbm, dst_hbm, out_hbm,
                src_all, dst_all, rows0, rows1, agg_sh,
                sg0, sg1, ss0, ss1):
    c = lax.axis_index("c")
    s = lax.axis_index("s")
    w = c * NS + s

    pltpu.sync_copy(src_hbm.at[w], src_all)
    pltpu.sync_copy(dst_hbm.at[w], dst_all)

    def _z(i, _):
        rows0[i // 8, pl.ds((i % 8) * 16, 16)] = jnp.zeros((16,), jnp.float32)
        return 0

    lax.fori_loop(0, K * 8, _z, 0)
    base = s * ROWS_PT
    for r in range(ROWS_PT // K):
        pltpu.sync_copy(rows0, agg_sh.at[pl.ds(base + r * K, K)])
    rem = ROWS_PT % K
    pltpu.sync_copy(rows0.at[pl.ds(0, rem)],
                    agg_sh.at[pl.ds(base + ROWS_PT - rem, rem)])
    plsc.subcore_barrier()

    def gather_start(j, rows, sem):
        pltpu.async_copy(g_hbm.at[src_all.at[pl.ds(j * K, K)]], rows, sem)

    def gather_wait(j, rows, sem):
        pltpu.make_async_copy(
            g_hbm.at[src_all.at[pl.ds(j * K, K)]], rows, sem).wait()

    def scatter_start(j, rows, sem):
        pltpu.async_copy(rows, agg_sh.at[dst_all.at[j]], sem, add=True)

    def scatter_wait(j, rows, sem):
        pltpu.make_async_copy(rows, agg_sh.at[dst_all.at[j]], sem).wait()

    gather_start(0, rows0, sg0)
    gather_start(1, rows1, sg1)

    def _step(t, _):
        a = 2 * t
        b = a + 1
        gather_wait(a, rows0, sg0)
        scatter_start(a, rows0, ss0)
        gather_wait(b, rows1, sg1)
        scatter_start(b, rows1, ss1)
        scatter_wait(a, rows0, ss0)
        gather_start(a + 2, rows0, sg0)

        @pl.when(t < NCHUNK // 2 - 2)
        def _():
            scatter_wait(b, rows1, ss1)
            gather_start(b + 2, rows1, sg1)

        return 0

    lax.fori_loop(0, NCHUNK // 2 - 1, _step, 0)

    gather_wait(NCHUNK - 2, rows0, sg0)
    scatter_start(NCHUNK - 2, rows0, ss0)
    scatter_wait(NCHUNK - 3, rows1, ss1)
    gather_start(NCHUNK - 1, rows1, sg1)
    gather_wait(NCHUNK - 1, rows1, sg1)
    scatter_start(NCHUNK - 1, rows1, ss1)
    scatter_wait(NCHUNK - 2, rows0, ss0)
    scatter_wait(NCHUNK - 1, rows1, ss1)

    plsc.subcore_barrier()
    pltpu.sync_copy(
        agg_sh.at[pl.ds(s * ROWS_PT, ROWS_PT)],
        out_hbm.at[c, pl.ds(s * ROWS_PT, ROWS_PT)],
    )


_R = 632
_RF = 1000


def _dinv(degA_ref, degB_ref):
    return lax.rsqrt(degA_ref[...] + degB_ref[...] + 1.0)


def _mm_body(x_ref, w_ref, o_ref):
    o_ref[...] = jnp.dot(x_ref[...], w_ref[...],
                         preferred_element_type=jnp.float32)


def _scale_body(h_ref, degA_ref, degB_ref, o_ref):
    o_ref[...] = h_ref[...] * _dinv(degA_ref, degB_ref)


def _layer2_body(g1_ref, aggA_ref, aggB_ref, degA_ref, degB_ref,
                 w2_ref, b1_ref, o_ref):
    dinv = _dinv(degA_ref, degB_ref)
    h = dinv * (aggA_ref[...] + aggB_ref[...] + g1_ref[...]) + b1_ref[...]
    h = jnp.maximum(h, 0.0)
    o_ref[...] = jnp.dot(h, w2_ref[...],
                         preferred_element_type=jnp.float32) * dinv


def _final_body(g2_ref, aggA_ref, aggB_ref, degA_ref, degB_ref,
                b2_ref, o_ref):
    dinv = _dinv(degA_ref, degB_ref)
    o_ref[...] = dinv * (aggA_ref[...] + aggB_ref[...] + g2_ref[...]) + b2_ref[...]


def _row_spec(r, width=D):
    return pl.BlockSpec((r, width), lambda i: (i, 0))


def _full_spec(shape):
    return pl.BlockSpec(shape, lambda i: (0, 0))


def _mm(xp, W):
    return pl.pallas_call(
        _mm_body,
        grid=(NP // _R,),
        in_specs=[_row_spec(_R), _full_spec((D, D))],
        out_specs=_row_spec(_R),
        out_shape=jax.ShapeDtypeStruct((NP, D), jnp.float32),
    )(xp, W)


def _scale(h, degA, degB):
    return pl.pallas_call(
        _scale_body,
        grid=(NP // _R,),
        in_specs=[_row_spec(_R), _row_spec(_R, 1), _row_spec(_R, 1)],
        out_specs=_row_spec(_R),
        out_shape=jax.ShapeDtypeStruct((NP, D), jnp.float32),
    )(h, degA, degB)


def _layer2(g1, aggA, aggB, degA, degB, W2, b1):
    return pl.pallas_call(
        _layer2_body,
        grid=(NP // _R,),
        in_specs=[_row_spec(_R), _row_spec(_R), _row_spec(_R),
                  _row_spec(_R, 1), _row_spec(_R, 1),
                  _full_spec((D, D)), _full_spec((1, D))],
        out_specs=_row_spec(_R),
        out_shape=jax.ShapeDtypeStruct((NP, D), jnp.float32),
    )(g1, aggA, aggB, degA, degB, W2, b1)


def _final(g2, aggA, aggB, degA, degB, b2):
    return pl.pallas_call(
        _final_body,
        grid=(N // _RF,),
        in_specs=[_row_spec(_RF), _row_spec(_RF), _row_spec(_RF),
                  _row_spec(_RF, 1), _row_spec(_RF, 1), _full_spec((1, D))],
        out_specs=_row_spec(_RF),
        out_shape=jax.ShapeDtypeStruct((N, D), jnp.float32),
    )(g2, aggA, aggB, degA, degB, b2)


def kernel(x, edge_index, W1, b1, W2, b2):
    src = edge_index[0].astype(jnp.int32)
    dst = edge_index[1].astype(jnp.int32)
    pad_idx = N + (jnp.arange(EP - E, dtype=jnp.int32) % (NP - N))
    src3 = jnp.concatenate([src, pad_idx]).reshape(NW, EPW)
    dst3 = jnp.concatenate([dst, pad_idx]).reshape(NW, NCHUNK, K)
    xp = jnp.pad(x, ((0, NP - N), (0, 0)))
    b1r = b1.reshape(1, D)
    b2r = b2.reshape(1, D)

    deg_parts = _deg_kernel(dst3)
    h1 = _mm(xp, W1)
    degA = deg_parts[0, :NP].reshape(NP, 1)
    degB = deg_parts[1, :NP].reshape(NP, 1)

    g1 = _scale(h1, degA, degB)
    agg1 = _agg_kernel(g1, src3, dst3)
    g2 = _layer2(g1, agg1[0], agg1[1], degA, degB, W2, b1r)
    agg2 = _agg_kernel(g2, src3, dst3)
    return _final(g2, agg2[0, :N], agg2[1, :N], degA[:N], degB[:N], b2r)


# speedup vs baseline: 2.3810x; 1.0031x over previous
import functools

import jax
import jax.numpy as jnp
from jax import lax
from jax.experimental import pallas as pl
from jax.experimental.pallas import tpu as pltpu
from jax.experimental.pallas import tpu_sc as plsc

N = 10000
D = 128
E = 320000
NC, NS = 2, 16
NW = NC * NS
K = 80
NCHUNK = 128
EPW = NCHUNK * K
EP = NW * EPW
NP = 10112
SINK = NP - 1
ROWS_PT = NP // NS
DEG_N = 10240
DEG_PT = DEG_N // NS

_MESH = dict(core_axis_name="c", subcore_axis_name="s")


@functools.partial(
    pl.kernel,
    out_type=jax.ShapeDtypeStruct((NC, DEG_N), jnp.float32),
    mesh=plsc.VectorSubcoreMesh(**_MESH),
    scratch_types=[
        pltpu.VMEM((8, K), jnp.int32),
        pltpu.VMEM((2, K), jnp.float32),
        pltpu.VMEM_SHARED((DEG_N,), jnp.float32),
    ],
)
def _deg_kernel(dst_hbm, out_hbm, dst_v, ones_v, deg_sh):
    c = lax.axis_index("c")
    s = lax.axis_index("s")
    w = c * NS + s

    for i in range(K // 16):
        ones_v[0, pl.ds(i * 16, 16)] = jnp.ones((16,), jnp.float32)
        ones_v[1, pl.ds(i * 16, 16)] = jnp.zeros((16,), jnp.float32)

    for r in range(DEG_PT // K):
        pltpu.sync_copy(ones_v.at[1],
                        deg_sh.at[pl.ds(s * DEG_PT + r * K, K)])
    plsc.subcore_barrier()

    def _slab(j, _):
        pltpu.sync_copy(dst_hbm.at[w, pl.ds(j * 8, 8)], dst_v)
        for r in range(8):
            pltpu.sync_copy(ones_v.at[0], deg_sh.at[dst_v.at[r]], add=True)
        return 0

    lax.fori_loop(0, NCHUNK // 8, _slab, 0)
    plsc.subcore_barrier()
    pltpu.sync_copy(
        deg_sh.at[pl.ds(s * DEG_PT, DEG_PT)],
        out_hbm.at[c, pl.ds(s * DEG_PT, DEG_PT)],
    )


@functools.partial(
    pl.kernel,
    out_type=jax.ShapeDtypeStruct((NC, NP, D), jnp.float32),
    mesh=plsc.VectorSubcoreMesh(**_MESH),
    scratch_types=[
        pltpu.VMEM((EPW,), jnp.int32),
        pltpu.VMEM((NCHUNK, K), jnp.int32),
        pltpu.VMEM((K, D), jnp.float32),
        pltpu.VMEM((K, D), jnp.float32),
        pltpu.VMEM_SHARED((NP, D), jnp.float32),
        pltpu.SemaphoreType.DMA,
        pltpu.SemaphoreType.DMA,
        pltpu.SemaphoreType.DMA,
        pltpu.SemaphoreType.DMA,
    ],
)
def _agg_kernel(g_hbm, src_hbm, dst_hbm, out_hbm,
                src_all, dst_all, rows0, rows1, agg_sh,
                sg0, sg1, ss0, ss1):
    c = lax.axis_index("c")
    s = lax.axis_index("s")
    w = c * NS + s

    pltpu.sync_copy(src_hbm.at[w], src_all)
    pltpu.sync_copy(dst_hbm.at[w], dst_all)

    def gather_start(j, rows, sem):
        pltpu.async_copy(g_hbm.at[src_all.at[pl.ds(j * K, K)]], rows, sem)

    def gather_wait(j, rows, sem):
        pltpu.make_async_copy(
            g_hbm.at[src_all.at[pl.ds(j * K, K)]], rows, sem).wait()

    def scatter_start(j, rows, sem):
        pltpu.async_copy(rows, agg_sh.at[dst_all.at[j]], sem, add=True)

    def scatter_wait(j, rows, sem):
        pltpu.make_async_copy(rows, agg_sh.at[dst_all.at[j]], sem).wait()

    def _z(i, _):
        rows0[i // 8, pl.ds((i % 8) * 16, 16)] = jnp.zeros((16,), jnp.float32)
        return 0

    lax.fori_loop(0, K * 8, _z, 0)
    base = s * ROWS_PT
    for r in range(ROWS_PT // K):
        pltpu.sync_copy(rows0, agg_sh.at[pl.ds(base + r * K, K)])
    rem = ROWS_PT % K
    pltpu.sync_copy(rows0.at[pl.ds(0, rem)],
                    agg_sh.at[pl.ds(base + ROWS_PT - rem, rem)])
    gather_start(0, rows0, sg0)
    gather_start(1, rows1, sg1)
    plsc.subcore_barrier()

    def _step(t, _):
        a = 2 * t
        b = a + 1
        gather_wait(a, rows0, sg0)
        scatter_start(a, rows0, ss0)
        gather_wait(b, rows1, sg1)
        scatter_start(b, rows1, ss1)
        scatter_wait(a, rows0, ss0)
        gather_start(a + 2, rows0, sg0)
        scatter_wait(b, rows1, ss1)
        gather_start(b + 2, rows1, sg1)
        return 0

    lax.fori_loop(0, NCHUNK // 2 - 1, _step, 0)

    gather_wait(NCHUNK - 2, rows0, sg0)
    scatter_start(NCHUNK - 2, rows0, ss0)
    gather_wait(NCHUNK - 1, rows1, sg1)
    scatter_start(NCHUNK - 1, rows1, ss1)
    scatter_wait(NCHUNK - 2, rows0, ss0)
    scatter_wait(NCHUNK - 1, rows1, ss1)

    plsc.subcore_barrier()
    pltpu.sync_copy(
        agg_sh.at[pl.ds(s * ROWS_PT, ROWS_PT)],
        out_hbm.at[c, pl.ds(s * ROWS_PT, ROWS_PT)],
    )


_R = 632
_RF = 1000


def _dinv(degA_ref, degB_ref):
    return lax.rsqrt(degA_ref[...] + degB_ref[...] + 1.0)


def _mm_body(x_ref, w_ref, o_ref):
    o_ref[...] = jnp.dot(x_ref[...], w_ref[...],
                         preferred_element_type=jnp.float32)


def _scale_body(h_ref, degA_ref, degB_ref, o_ref):
    o_ref[...] = h_ref[...] * _dinv(degA_ref, degB_ref)


def _layer2_body(g1_ref, aggA_ref, aggB_ref, degA_ref, degB_ref,
                 w2_ref, b1_ref, o_ref):
    dinv = _dinv(degA_ref, degB_ref)
    h = dinv * (aggA_ref[...] + aggB_ref[...] + g1_ref[...]) + b1_ref[...]
    h = jnp.maximum(h, 0.0)
    o_ref[...] = jnp.dot(h, w2_ref[...],
                         preferred_element_type=jnp.float32) * dinv


def _final_body(g2_ref, aggA_ref, aggB_ref, degA_ref, degB_ref,
                b2_ref, o_ref):
    dinv = _dinv(degA_ref, degB_ref)
    o_ref[...] = dinv * (aggA_ref[...] + aggB_ref[...] + g2_ref[...]) + b2_ref[...]


def _row_spec(r, width=D):
    return pl.BlockSpec((r, width), lambda i: (i, 0))


def _full_spec(shape):
    return pl.BlockSpec(shape, lambda i: (0, 0))


def _mm(xp, W):
    return pl.pallas_call(
        _mm_body,
        grid=(NP // _R,),
        in_specs=[_row_spec(_R), _full_spec((D, D))],
        out_specs=_row_spec(_R),
        out_shape=jax.ShapeDtypeStruct((NP, D), jnp.float32),
    )(xp, W)


def _scale(h, degA, degB):
    return pl.pallas_call(
        _scale_body,
        grid=(NP // _R,),
        in_specs=[_row_spec(_R), _row_spec(_R, 1), _row_spec(_R, 1)],
        out_specs=_row_spec(_R),
        out_shape=jax.ShapeDtypeStruct((NP, D), jnp.float32),
    )(h, degA, degB)


def _layer2(g1, aggA, aggB, degA, degB, W2, b1):
    return pl.pallas_call(
        _layer2_body,
        grid=(NP // _R,),
        in_specs=[_row_spec(_R), _row_spec(_R), _row_spec(_R),
                  _row_spec(_R, 1), _row_spec(_R, 1),
                  _full_spec((D, D)), _full_spec((1, D))],
        out_specs=_row_spec(_R),
        out_shape=jax.ShapeDtypeStruct((NP, D), jnp.float32),
    )(g1, aggA, aggB, degA, degB, W2, b1)


def _final(g2, aggA, aggB, degA, degB, b2):
    return pl.pallas_call(
        _final_body,
        grid=(N // _RF,),
        in_specs=[_row_spec(_RF), _row_spec(_RF), _row_spec(_RF),
                  _row_spec(_RF, 1), _row_spec(_RF, 1), _full_spec((1, D))],
        out_specs=_row_spec(_RF),
        out_shape=jax.ShapeDtypeStruct((N, D), jnp.float32),
    )(g2, aggA, aggB, degA, degB, b2)


def kernel(x, edge_index, W1, b1, W2, b2):
    src = edge_index[0].astype(jnp.int32)
    dst = edge_index[1].astype(jnp.int32)
    pad_idx = N + (jnp.arange(EP - E, dtype=jnp.int32) % (NP - N))
    src3 = jnp.concatenate([src, pad_idx]).reshape(NW, EPW)
    dst3 = jnp.concatenate([dst, pad_idx]).reshape(NW, NCHUNK, K)
    xp = jnp.pad(x, ((0, NP - N), (0, 0)))
    b1r = b1.reshape(1, D)
    b2r = b2.reshape(1, D)

    deg_parts = _deg_kernel(dst3)
    h1 = _mm(xp, W1)
    degA = deg_parts[0, :NP].reshape(NP, 1)
    degB = deg_parts[1, :NP].reshape(NP, 1)

    g1 = _scale(h1, degA, degB)
    agg1 = _agg_kernel(g1, src3, dst3)
    g2 = _layer2(g1, agg1[0], agg1[1], degA, degB, W2, b1r)
    agg2 = _agg_kernel(g2, src3, dst3)
    return _final(g2, agg2[0, :N], agg2[1, :N], degA[:N], degB[:N], b2r)


# speedup vs baseline: 2.3847x; 1.0016x over previous
import functools

import jax
import jax.numpy as jnp
from jax import lax
from jax.experimental import pallas as pl
from jax.experimental.pallas import tpu as pltpu
from jax.experimental.pallas import tpu_sc as plsc

N = 10000
D = 128
E = 320000
NC, NS = 2, 16
NW = NC * NS
K = 80
NCHUNK = 128
EPW = NCHUNK * K
EP = NW * EPW
NP = 10112
SINK = NP - 1
ROWS_PT = NP // NS
DEG_N = 10240
DEG_PT = DEG_N // NS

_MESH = dict(core_axis_name="c", subcore_axis_name="s")


@functools.partial(
    pl.kernel,
    out_type=jax.ShapeDtypeStruct((NC, DEG_N), jnp.float32),
    mesh=plsc.VectorSubcoreMesh(**_MESH),
    scratch_types=[
        pltpu.VMEM((8, K), jnp.int32),
        pltpu.VMEM((2, K), jnp.float32),
        pltpu.VMEM_SHARED((DEG_N,), jnp.float32),
    ],
)
def _deg_kernel(dst_hbm, out_hbm, dst_v, ones_v, deg_sh):
    c = lax.axis_index("c")
    s = lax.axis_index("s")
    w = c * NS + s

    for i in range(K // 16):
        ones_v[0, pl.ds(i * 16, 16)] = jnp.ones((16,), jnp.float32)
        ones_v[1, pl.ds(i * 16, 16)] = jnp.zeros((16,), jnp.float32)

    for r in range(DEG_PT // K):
        pltpu.sync_copy(ones_v.at[1],
                        deg_sh.at[pl.ds(s * DEG_PT + r * K, K)])
    plsc.subcore_barrier()

    def _slab(j, _):
        pltpu.sync_copy(dst_hbm.at[w, pl.ds(j * 8, 8)], dst_v)
        for r in range(8):
            pltpu.sync_copy(ones_v.at[0], deg_sh.at[dst_v.at[r]], add=True)
        return 0

    lax.fori_loop(0, NCHUNK // 8, _slab, 0)
    plsc.subcore_barrier()
    pltpu.sync_copy(
        deg_sh.at[pl.ds(s * DEG_PT, DEG_PT)],
        out_hbm.at[c, pl.ds(s * DEG_PT, DEG_PT)],
    )


@functools.partial(
    pl.kernel,
    out_type=jax.ShapeDtypeStruct((NC, NP, D), jnp.float32),
    mesh=plsc.VectorSubcoreMesh(**_MESH),
    scratch_types=[
        pltpu.VMEM((EPW,), jnp.int32),
        pltpu.VMEM((NCHUNK, K), jnp.int32),
        pltpu.VMEM((K, D), jnp.float32),
        pltpu.VMEM((K, D), jnp.float32),
        pltpu.VMEM_SHARED((NP, D), jnp.float32),
        pltpu.SemaphoreType.DMA,
        pltpu.SemaphoreType.DMA,
        pltpu.SemaphoreType.DMA,
        pltpu.SemaphoreType.DMA,
    ],
)
def _agg_kernel(g_hbm, src_hbm, dst_hbm, out_hbm,
                src_all, dst_all, rows0, rows1, agg_sh,
                sg0, sg1, ss0, ss1):
    c = lax.axis_index("c")
    s = lax.axis_index("s")
    w = c * NS + s

    pltpu.sync_copy(src_hbm.at[w], src_all)
    pltpu.sync_copy(dst_hbm.at[w], dst_all)

    H = K // 2

    def gather_start(j, rows, sem):
        pltpu.async_copy(g_hbm.at[src_all.at[pl.ds(j * K, H)]],
                         rows.at[pl.ds(0, H)], sem)
        pltpu.async_copy(g_hbm.at[src_all.at[pl.ds(j * K + H, H)]],
                         rows.at[pl.ds(H, H)], sem)

    def gather_wait(j, rows, sem):
        pltpu.make_async_copy(
            g_hbm.at[src_all.at[pl.ds(j * K, K)]], rows, sem).wait()

    def scatter_start(j, rows, sem):
        pltpu.async_copy(rows, agg_sh.at[dst_all.at[j]], sem, add=True)

    def scatter_wait(j, rows, sem):
        pltpu.make_async_copy(rows, agg_sh.at[dst_all.at[j]], sem).wait()

    def _z(i, _):
        rows0[i // 8, pl.ds((i % 8) * 16, 16)] = jnp.zeros((16,), jnp.float32)
        return 0

    lax.fori_loop(0, K * 8, _z, 0)
    base = s * ROWS_PT
    for r in range(ROWS_PT // K):
        pltpu.sync_copy(rows0, agg_sh.at[pl.ds(base + r * K, K)])
    rem = ROWS_PT % K
    pltpu.sync_copy(rows0.at[pl.ds(0, rem)],
                    agg_sh.at[pl.ds(base + ROWS_PT - rem, rem)])
    gather_start(0, rows0, sg0)
    gather_start(1, rows1, sg1)
    plsc.subcore_barrier()

    def _step(t, _):
        a = 2 * t
        b = a + 1
        gather_wait(a, rows0, sg0)
        scatter_start(a, rows0, ss0)
        gather_wait(b, rows1, sg1)
        scatter_start(b, rows1, ss1)
        scatter_wait(a, rows0, ss0)
        gather_start(a + 2, rows0, sg0)
        scatter_wait(b, rows1, ss1)
        gather_start(b + 2, rows1, sg1)
        return 0

    lax.fori_loop(0, NCHUNK // 2 - 1, _step, 0)

    gather_wait(NCHUNK - 2, rows0, sg0)
    scatter_start(NCHUNK - 2, rows0, ss0)
    gather_wait(NCHUNK - 1, rows1, sg1)
    scatter_start(NCHUNK - 1, rows1, ss1)
    scatter_wait(NCHUNK - 2, rows0, ss0)
    scatter_wait(NCHUNK - 1, rows1, ss1)

    plsc.subcore_barrier()
    pltpu.sync_copy(
        agg_sh.at[pl.ds(s * ROWS_PT, ROWS_PT)],
        out_hbm.at[c, pl.ds(s * ROWS_PT, ROWS_PT)],
    )


_R = 632
_RF = 1000


def _dinv(degA_ref, degB_ref):
    return lax.rsqrt(degA_ref[...] + degB_ref[...] + 1.0)


def _mm_body(x_ref, w_ref, o_ref):
    o_ref[...] = jnp.dot(x_ref[...], w_ref[...],
                         preferred_element_type=jnp.float32)


def _scale_body(h_ref, degA_ref, degB_ref, o_ref):
    o_ref[...] = h_ref[...] * _dinv(degA_ref, degB_ref)


def _layer2_body(g1_ref, aggA_ref, aggB_ref, degA_ref, degB_ref,
                 w2_ref, b1_ref, o_ref):
    dinv = _dinv(degA_ref, degB_ref)
    h = dinv * (aggA_ref[...] + aggB_ref[...] + g1_ref[...]) + b1_ref[...]
    h = jnp.maximum(h, 0.0)
    o_ref[...] = jnp.dot(h, w2_ref[...],
                         preferred_element_type=jnp.float32) * dinv


def _final_body(g2_ref, aggA_ref, aggB_ref, degA_ref, degB_ref,
                b2_ref, o_ref):
    dinv = _dinv(degA_ref, degB_ref)
    o_ref[...] = dinv * (aggA_ref[...] + aggB_ref[...] + g2_ref[...]) + b2_ref[...]


def _row_spec(r, width=D):
    return pl.BlockSpec((r, width), lambda i: (i, 0))


def _full_spec(shape):
    return pl.BlockSpec(shape, lambda i: (0, 0))


def _mm(xp, W):
    return pl.pallas_call(
        _mm_body,
        grid=(NP // _R,),
        in_specs=[_row_spec(_R), _full_spec((D, D))],
        out_specs=_row_spec(_R),
        out_shape=jax.ShapeDtypeStruct((NP, D), jnp.float32),
    )(xp, W)


def _scale(h, degA, degB):
    return pl.pallas_call(
        _scale_body,
        grid=(NP // _R,),
        in_specs=[_row_spec(_R), _row_spec(_R, 1), _row_spec(_R, 1)],
        out_specs=_row_spec(_R),
        out_shape=jax.ShapeDtypeStruct((NP, D), jnp.float32),
    )(h, degA, degB)


def _layer2(g1, aggA, aggB, degA, degB, W2, b1):
    return pl.pallas_call(
        _layer2_body,
        grid=(NP // _R,),
        in_specs=[_row_spec(_R), _row_spec(_R), _row_spec(_R),
                  _row_spec(_R, 1), _row_spec(_R, 1),
                  _full_spec((D, D)), _full_spec((1, D))],
        out_specs=_row_spec(_R),
        out_shape=jax.ShapeDtypeStruct((NP, D), jnp.float32),
    )(g1, aggA, aggB, degA, degB, W2, b1)


def _final(g2, aggA, aggB, degA, degB, b2):
    return pl.pallas_call(
        _final_body,
        grid=(N // _RF,),
        in_specs=[_row_spec(_RF), _row_spec(_RF), _row_spec(_RF),
                  _row_spec(_RF, 1), _row_spec(_RF, 1), _full_spec((1, D))],
        out_specs=_row_spec(_RF),
        out_shape=jax.ShapeDtypeStruct((N, D), jnp.float32),
    )(g2, aggA, aggB, degA, degB, b2)


def kernel(x, edge_index, W1, b1, W2, b2):
    src = edge_index[0].astype(jnp.int32)
    dst = edge_index[1].astype(jnp.int32)
    pad_idx = N + (jnp.arange(EP - E, dtype=jnp.int32) % (NP - N))
    src3 = jnp.concatenate([src, pad_idx]).reshape(NW, EPW)
    dst3 = jnp.concatenate([dst, pad_idx]).reshape(NW, NCHUNK, K)
    xp = jnp.pad(x, ((0, NP - N), (0, 0)))
    b1r = b1.reshape(1, D)
    b2r = b2.reshape(1, D)

    deg_parts = _deg_kernel(dst3)
    h1 = _mm(xp, W1)
    degA = deg_parts[0, :NP].reshape(NP, 1)
    degB = deg_parts[1, :NP].reshape(NP, 1)

    g1 = _scale(h1, degA, degB)
    agg1 = _agg_kernel(g1, src3, dst3)
    g2 = _layer2(g1, agg1[0], agg1[1], degA, degB, W2, b1r)
    agg2 = _agg_kernel(g2, src3, dst3)
    return _final(g2, agg2[0, :N], agg2[1, :N], degA[:N], degB[:N], b2r)
